# baseline (speedup 1.0000x reference)
import functools
import math

import jax
import jax.numpy as jnp
from jax import lax
from jax.experimental import pallas as pl
from jax.experimental.pallas import tpu as pltpu
from jax.experimental.pallas import tpu_sc as plsc

V = 32768
N = 4096
PAD = 0
EPS = 0.1 / (V - 2)
COEF = 1.0 - 0.1 - EPS
CONST = 0.9 * math.log(0.9) + 0.1 * math.log(EPS)

R = 128
NR = N // R


def _tc_body(yhat_ref, tgt_ref, out_ref):
    r = pl.program_id(0)
    acc = jnp.sum(yhat_ref[...], axis=1, keepdims=True) - yhat_ref[:, 0:1]
    mask = tgt_ref[0] != PAD
    total = jnp.sum(jnp.where(mask, -EPS * acc, 0.0)).reshape(1, 1)

    @pl.when(r == 0)
    def _():
        out_ref[...] = total

    @pl.when(r > 0)
    def _():
        out_ref[...] += total


def _tc_dense(yhat, tgt3):
    out = pl.pallas_call(
        _tc_body,
        grid=(NR,),
        in_specs=[
            pl.BlockSpec((R, V), lambda r: (r, 0)),
            pl.BlockSpec((1, R, 1), lambda r: (r, 0, 0)),
        ],
        out_specs=pl.BlockSpec((1, 1), lambda r: (0, 0)),
        out_shape=jax.ShapeDtypeStruct((1, 1), jnp.float32),
        compiler_params=pltpu.CompilerParams(
            dimension_semantics=("arbitrary",)),
    )(yhat, tgt3)
    return out[0, 0]


_INFO = plsc.get_sparse_core_info()
NC, NS, L = _INFO.num_cores, _INFO.num_subcores, _INFO.num_lanes
NW = NC * NS
RPW = N // NW
NG = RPW // L

_SC_MESH = plsc.VectorSubcoreMesh(core_axis_name="c", subcore_axis_name="s")


@functools.partial(
    pl.kernel,
    mesh=_SC_MESH,
    out_type=jax.ShapeDtypeStruct((NW, L), jnp.float32),
    scratch_types=[
        pltpu.VMEM((RPW,), jnp.int32),
        pltpu.VMEM((RPW * L,), jnp.float32),
        pltpu.VMEM((L,), jnp.float32),
        pltpu.SemaphoreType.DMA,
    ],
)
def _sc_pick(yhat_hbm, tgt_hbm, out_hbm, tgt_v, wbuf, out_v, sem):
    wid = lax.axis_index("s") * NC + lax.axis_index("c")
    row_base = wid * RPW
    pltpu.sync_copy(tgt_hbm.at[pl.ds(row_base, RPW)], tgt_v)
    lane = lax.iota(jnp.int32, L)
    accp = jnp.zeros((L,), jnp.float32)
    cnt = jnp.zeros((L,), jnp.float32)
    tvecs = [tgt_v[pl.ds(g * L, L)] for g in range(NG)]
    offs = [[(tvecs[g][j] // L) * L for j in range(L)] for g in range(NG)]
    handles = []
    for g in range(NG):
        for j in range(L):
            handles.append(pltpu.async_copy(
                yhat_hbm.at[row_base + g * L + j, pl.ds(offs[g][j], L)],
                wbuf.at[pl.ds((g * L + j) * L, L)], sem))
    for h in handles:
        h.wait()
    for g in range(NG):
        tvec = tvecs[g]
        for j in range(L):
            tadj = jnp.where(tvec[j] == PAD, -1, tvec[j])
            w = wbuf[pl.ds((g * L + j) * L, L)]
            hit = (lane + (offs[g][j] - tadj)) == 0
            accp = accp + jnp.where(hit, w, 0.0)
        cnt = cnt + jnp.where(tvec != PAD, 1.0, 0.0)
    out_v[...] = CONST * cnt - COEF * accp
    pltpu.sync_copy(out_v, out_hbm.at[wid])


def kernel(yhat, target):
    sc_partials = _sc_pick(yhat, target)
    tc_part = _tc_dense(yhat, target.reshape(NR, R, 1))
    return tc_part + jnp.sum(sc_partials)
